# baseline (speedup 1.0000x reference)
import functools

import jax
import jax.numpy as jnp
from jax import lax
from jax.experimental import pallas as pl
from jax.experimental.pallas import tpu as pltpu
from jax.experimental.pallas import tpu_sc as plsc

_WIDTH = 1000
_FEATURE_DIM = 100000
_N = 1024

_INFO = plsc.get_sparse_core_info()
_NW = _INFO.num_cores * _INFO.num_subcores
_ROWS_PER = _N // _NW
_CHUNKS = _ROWS_PER // 16

_SEG = 25600
_STARTS = (0, _SEG, 2 * _SEG, 3 * _SEG)
_LENS = (_SEG, _SEG, _SEG, _FEATURE_DIM - 3 * _SEG)


def _sc_onehot_body(x_hbm, y_hbm, out_hbm, b0, b1, b2, b3, xbuf, ybuf, sems):
    bufs = (b0, b1, b2, b3)
    wid = lax.axis_index("s") * _INFO.num_cores + lax.axis_index("c")
    base = wid * _ROWS_PER

    lanes = lax.broadcasted_iota(jnp.int32, (16,), 0)
    zeros_f = jnp.zeros((16,), jnp.float32)

    def zero_body(i, carry):
        off = pl.ds(pl.multiple_of(i * 16, 16), 16)
        for b in bufs:
            b[off] = zeros_f
        return carry

    lax.fori_loop(0, _SEG // 16, zero_body, 0, unroll=4)

    def seg_dma(s, row):
        return pltpu.make_async_copy(
            bufs[s].at[pl.ds(0, _LENS[s])],
            out_hbm.at[row, pl.ds(_STARTS[s], _LENS[s])],
            sems.at[s],
        )

    prev = None
    for k in range(_CHUNKS):
        cbase = base + k * 16
        pltpu.sync_copy(x_hbm.at[pl.ds(cbase, 16)], xbuf)
        pltpu.sync_copy(y_hbm.at[pl.ds(cbase, 16)], ybuf)
        idxv = xbuf[...] + _WIDTH * ybuf[...]

        pos = []
        val = []
        for s in range(4):
            rel = idxv - _STARTS[s]
            in_seg = (rel >= 0) & (rel < _LENS[s])
            pos.append(jnp.clip(rel, 0, _LENS[s] - 1))
            val.append(jnp.where(in_seg, 1.0, 0.0).astype(jnp.float32))

        for l in range(16):
            r = k * 16 + l
            mask = lanes == l
            for s in range(4):
                if prev is not None:
                    pmask, ppos = prev[0], prev[1][s]
                    seg_dma(s, base + r - 1).wait()
                    plsc.store_scatter(bufs[s], [ppos], zeros_f, mask=pmask)
                plsc.store_scatter(bufs[s], [pos[s]], val[s], mask=mask)
                seg_dma(s, base + r).start()
            prev = (mask, pos)

    for s in range(4):
        seg_dma(s, base + _ROWS_PER - 1).wait()


def kernel(state):
    mesh = plsc.VectorSubcoreMesh(core_axis_name="c", subcore_axis_name="s")
    sc_onehot = functools.partial(
        pl.kernel,
        mesh=mesh,
        out_type=jax.ShapeDtypeStruct((_N, _FEATURE_DIM), jnp.float32),
        scratch_types=[
            pltpu.VMEM((_SEG,), jnp.float32),
            pltpu.VMEM((_SEG,), jnp.float32),
            pltpu.VMEM((_SEG,), jnp.float32),
            pltpu.VMEM((_SEG,), jnp.float32),
            pltpu.VMEM((16,), jnp.int32),
            pltpu.VMEM((16,), jnp.int32),
            pltpu.SemaphoreType.DMA((4,)),
        ],
        compiler_params=pltpu.CompilerParams(
            needs_layout_passes=False, use_tc_tiling_on_sc=False
        ),
    )(_sc_onehot_body)
    return sc_onehot(state[:, 0], state[:, 1])


# speedup vs baseline: 2.0041x; 2.0041x over previous
import functools

import jax
import jax.numpy as jnp
from jax import lax
from jax.experimental import pallas as pl
from jax.experimental.pallas import tpu as pltpu
from jax.experimental.pallas import tpu_sc as plsc

_WIDTH = 1000
_FEATURE_DIM = 100000
_N = 1024

_INFO = plsc.get_sparse_core_info()
_NW = _INFO.num_cores * _INFO.num_subcores
_ROWS_PER = _N // _NW
_CHUNKS = _ROWS_PER // 16

_SEG = 25600
_STARTS = (0, _SEG, 2 * _SEG, 3 * _SEG)
_TAIL0 = (_FEATURE_DIM // 128) * 128
_LENS = (_SEG, _SEG, _SEG, _TAIL0 - 3 * _SEG)


def _sc_onehot_body(x_hbm, y_hbm, out_hbm, b0, b1, b2, b3, xbuf, ybuf, sems):
    bufs = (b0, b1, b2, b3)
    wid = lax.axis_index("s") * _INFO.num_cores + lax.axis_index("c")
    base = wid * _ROWS_PER

    lanes = lax.broadcasted_iota(jnp.int32, (16,), 0)
    zeros_f = jnp.zeros((16,), jnp.float32)

    def zero_body(i, carry):
        off = pl.ds(pl.multiple_of(i * 16, 16), 16)
        for b in bufs:
            b[off] = zeros_f
        return carry

    lax.fori_loop(0, _SEG // 16, zero_body, 0, unroll=4)

    def seg_dma(s, row):
        return pltpu.make_async_copy(
            bufs[s].at[pl.ds(0, _LENS[s])],
            out_hbm.at[row, pl.ds(_STARTS[s], _LENS[s])],
            sems.at[s],
        )

    prev = None
    for k in range(_CHUNKS):
        cbase = base + k * 16
        pltpu.sync_copy(x_hbm.at[pl.ds(cbase, 16)], xbuf)
        pltpu.sync_copy(y_hbm.at[pl.ds(cbase, 16)], ybuf)
        idxv = xbuf[...] + _WIDTH * ybuf[...]

        pos = []
        val = []
        for s in range(4):
            rel = idxv - _STARTS[s]
            in_seg = (rel >= 0) & (rel < _LENS[s])
            pos.append(jnp.clip(rel, 0, _LENS[s] - 1))
            val.append(jnp.where(in_seg, 1.0, 0.0).astype(jnp.float32))

        for l in range(16):
            r = k * 16 + l
            mask = lanes == l
            for s in range(4):
                if prev is not None:
                    pmask, ppos = prev[0], prev[1][s]
                    seg_dma(s, base + r - 1).wait()
                    plsc.store_scatter(bufs[s], [ppos], zeros_f, mask=pmask)
                plsc.store_scatter(bufs[s], [pos[s]], val[s], mask=mask)
                seg_dma(s, base + r).start()
            prev = (mask, pos)

    for s in range(4):
        seg_dma(s, base + _ROWS_PER - 1).wait()


def kernel(state):
    mesh = plsc.VectorSubcoreMesh(core_axis_name="c", subcore_axis_name="s")
    sc_onehot = functools.partial(
        pl.kernel,
        mesh=mesh,
        out_type=jax.ShapeDtypeStruct((_N, _FEATURE_DIM), jnp.float32),
        scratch_types=[
            pltpu.VMEM((_SEG,), jnp.float32),
            pltpu.VMEM((_SEG,), jnp.float32),
            pltpu.VMEM((_SEG,), jnp.float32),
            pltpu.VMEM((_SEG,), jnp.float32),
            pltpu.VMEM((16,), jnp.int32),
            pltpu.VMEM((16,), jnp.int32),
            pltpu.SemaphoreType.DMA((4,)),
        ],
        compiler_params=pltpu.CompilerParams(needs_layout_passes=False),
    )(_sc_onehot_body)
    partial = sc_onehot(state[:, 0], state[:, 1])
    return pl.pallas_call(
        _tail_zero_body,
        in_specs=[pl.BlockSpec(memory_space=pl.ANY)],
        out_specs=pl.BlockSpec(memory_space=pl.ANY),
        out_shape=jax.ShapeDtypeStruct((_N, _FEATURE_DIM), jnp.float32),
        input_output_aliases={0: 0},
        scratch_shapes=[
            pltpu.VMEM((_N, _FEATURE_DIM - _TAIL0), jnp.float32),
            pltpu.SemaphoreType.DMA,
        ],
    )(partial)


def _tail_zero_body(prev_hbm, out_hbm, zbuf, sem):
    del prev_hbm
    zbuf[...] = jnp.zeros_like(zbuf)
    cp = pltpu.make_async_copy(
        zbuf,
        out_hbm.at[:, pl.ds(_TAIL0, _FEATURE_DIM - _TAIL0)],
        sem,
    )
    cp.start()
    cp.wait()


# speedup vs baseline: 2.0263x; 1.0111x over previous
import functools

import jax
import jax.numpy as jnp
from jax import lax
from jax.experimental import pallas as pl
from jax.experimental.pallas import tpu as pltpu
from jax.experimental.pallas import tpu_sc as plsc

_WIDTH = 1000
_FEATURE_DIM = 100000
_N = 1024

_INFO = plsc.get_sparse_core_info()
_NW = _INFO.num_cores * _INFO.num_subcores
_ROWS_PER = _N // _NW
_CHUNKS = _ROWS_PER // 16


def _sc_onehot_body(x_hbm, y_hbm, out_hbm, rowbuf, xbuf, ybuf):
    wid = lax.axis_index("s") * _INFO.num_cores + lax.axis_index("c")
    base = wid * _ROWS_PER

    lanes = lax.broadcasted_iota(jnp.int32, (16,), 0)
    ones_f = jnp.ones((16,), jnp.float32)
    zeros_f = jnp.zeros((16,), jnp.float32)

    def zero_body(i, carry):
        rowbuf[pl.ds(pl.multiple_of(i * 16, 16), 16)] = zeros_f
        return carry

    lax.fori_loop(0, _FEATURE_DIM // 16, zero_body, 0, unroll=8)

    for k in range(_CHUNKS):
        cbase = base + k * 16
        pltpu.sync_copy(x_hbm.at[pl.ds(cbase, 16)], xbuf)
        pltpu.sync_copy(y_hbm.at[pl.ds(cbase, 16)], ybuf)
        idxv = xbuf[...] + _WIDTH * ybuf[...]

        def row_body(l, idxv):
            mask = lanes == l
            plsc.store_scatter(rowbuf, [idxv], ones_f, mask=mask)
            pltpu.sync_copy(rowbuf, out_hbm.at[cbase + l])
            plsc.store_scatter(rowbuf, [idxv], zeros_f, mask=mask)
            return idxv

        lax.fori_loop(0, 16, row_body, idxv)


def kernel(state):
    mesh = plsc.VectorSubcoreMesh(core_axis_name="c", subcore_axis_name="s")
    sc_onehot = functools.partial(
        pl.kernel,
        mesh=mesh,
        out_type=jax.ShapeDtypeStruct((_N, _FEATURE_DIM), jnp.float32),
        scratch_types=[
            pltpu.VMEM((_FEATURE_DIM,), jnp.float32),
            pltpu.VMEM((16,), jnp.int32),
            pltpu.VMEM((16,), jnp.int32),
        ],
        compiler_params=pltpu.CompilerParams(needs_layout_passes=False),
    )(_sc_onehot_body)
    return sc_onehot(state[:, 0], state[:, 1])
